# baseline (speedup 1.0000x reference)
import functools
import math

import jax
import jax.numpy as jnp
from jax import lax
from jax.experimental import pallas as pl
from jax.experimental.pallas import tpu as pltpu
from jax.experimental.pallas import tpu_sc as plsc

TOKEN_SELF_ATTN_VALUE = -5e4
BUCKET = 64
N_HASHES = 8
T = 2048
E = 64
BH = 12
NB = T // BUCKET
NHB = N_HASHES * BH
ROWS = NHB * T
SRC_ROWS = BH * T

_HIGH = jax.lax.Precision.HIGHEST


def _stage_a_body(qk_ref, rot_ref, rank_ref):
    bh = pl.program_id(0)
    x = qk_ref[0]
    r = jax.lax.dot(x, rot_ref[...],
                    preferred_element_type=jnp.float32)

    lane32 = jax.lax.broadcasted_iota(jnp.int32, (T, 32), 1)
    ohs = []
    for h in range(N_HASHES):
        rh = r[:, 16 * h:16 * h + 16]
        r2 = jnp.concatenate([rh, -rh], axis=1)
        mx = jnp.max(r2, axis=1, keepdims=True)
        cand = jnp.where(r2 >= mx, lane32, 32)
        bucket = jnp.min(cand, axis=1, keepdims=True)
        ohs.append((lane32 == bucket).astype(jnp.float32))
    oh = jnp.concatenate(ohs, axis=1)

    ri = jax.lax.broadcasted_iota(jnp.int32, (128, 128), 0)
    ci = jax.lax.broadcasted_iota(jnp.int32, (128, 128), 1)
    tri = (ri >= ci).astype(jnp.float32)
    parts = []
    offs = jnp.zeros((1, 256), jnp.float32)
    for blk in range(T // 128):
        xb = oh[128 * blk:128 * (blk + 1), :]
        inc = jax.lax.dot(tri, xb,
                          preferred_element_type=jnp.float32)
        parts.append(inc + offs)
        offs = offs + inc[127:128, :]
    csum = jnp.concatenate(parts, axis=0)
    totals = offs

    gi = jax.lax.broadcasted_iota(jnp.int32, (256, 256), 0)
    gj = jax.lax.broadcasted_iota(jnp.int32, (256, 256), 1)
    same_h = (gi // 32) == (gj // 32)
    before = jnp.logical_and(same_h, gi < gj).astype(jnp.float32)
    start = jax.lax.dot(totals, before, precision=_HIGH,
                        preferred_element_type=jnp.float32)

    sel = oh * (csum + start)
    rank = jnp.concatenate(
        [jnp.sum(sel[:, 32 * h:32 * h + 32], axis=1, keepdims=True)
         for h in range(N_HASHES)], axis=1) - 1.0

    h_iota = jax.lax.broadcasted_iota(jnp.int32, (T, N_HASHES), 1)
    base = h_iota * (BH * T) + bh * T
    rank_ref[0] = rank.astype(jnp.int32) + base


def _hash_rank(qkf, rot):
    return pl.pallas_call(
        _stage_a_body,
        grid=(BH,),
        in_specs=[
            pl.BlockSpec((1, T, E), lambda b: (b, 0, 0)),
            pl.BlockSpec((E, 16 * N_HASHES), lambda b: (0, 0)),
        ],
        out_specs=pl.BlockSpec((1, T, N_HASHES), lambda b: (b, 0, 0)),
        out_shape=jax.ShapeDtypeStruct((BH, T, N_HASHES), jnp.int32),
    )(qkf, rot)


def _sc_mesh():
    return plsc.VectorSubcoreMesh(core_axis_name="c", subcore_axis_name="s")


def _sort_scatter(qv, idx2):
    info = plsc.get_sparse_core_info()
    nw = info.num_cores * info.num_subcores
    per_w = ROWS // nw
    ch = 1024
    n_ch = per_w // ch

    @functools.partial(
        pl.kernel,
        mesh=_sc_mesh(),
        out_type=jax.ShapeDtypeStruct((ROWS, 2 * E), jnp.float32),
        scratch_types=[
            pltpu.VMEM((8, 128), jnp.int32),
            pltpu.VMEM((ch // 2, 2 * E), jnp.float32),
            pltpu.SemaphoreType.DMA,
        ],
    )
    def scatter_k(qv_hbm, idx_hbm, out_hbm, idx_v, qr, sem):
        wid = lax.axis_index("s") * info.num_cores + lax.axis_index("c")
        base = wid * per_w
        sbase = (wid % (nw // N_HASHES)) * per_w

        def chunk(c, _):
            off = pl.multiple_of(base + c * ch, ch)
            soff = pl.multiple_of(sbase + c * ch, ch)
            pltpu.sync_copy(idx_hbm.at[pl.ds(pl.multiple_of(off // 128, 8), 8)],
                            idx_v)
            for half in range(2):
                hoff = pl.multiple_of(soff + half * (ch // 2), ch // 2)
                pltpu.sync_copy(qv_hbm.at[pl.ds(hoff, ch // 2)], qr)
                cps = []
                for s in range(ch // 2 // 128):
                    row = half * (ch // 2 // 128) + s
                    cps.append(pltpu.async_copy(
                        qr.at[pl.ds(s * 128, 128)],
                        out_hbm.at[idx_v.at[row]], sem))
                for cp in cps:
                    cp.wait()
            return _

        lax.fori_loop(0, n_ch, chunk, None)

    return scatter_k(qv, idx2)


_GRP = 4
_QM = _GRP * BUCKET
_KM = (_GRP + 1) * BUCKET


def _stage_c_body(sqv_ref, out_ref, knx, vx):
    temp = 1.0 / math.sqrt(E)
    q = sqv_ref[0, :, :E]
    v = sqv_ref[0, :, E:]
    ss = jnp.sum(q * q, axis=1, keepdims=True)
    inv_norm = 1.0 / jnp.maximum(jnp.sqrt(ss), 1e-12)
    kn = q * inv_norm
    knx[pl.ds(0, BUCKET), :] = kn[T - BUCKET:, :]
    knx[pl.ds(BUCKET, T), :] = kn
    vx[pl.ds(0, BUCKET), :] = v[T - BUCKET:, :]
    vx[pl.ds(BUCKET, T), :] = v

    ii = jax.lax.broadcasted_iota(jnp.int32, (_QM, 2 * BUCKET), 0)
    jj = jax.lax.broadcasted_iota(jnp.int32, (_QM, 2 * BUCKET), 1)
    self_add = jnp.where(jj == BUCKET + ii % BUCKET,
                         TOKEN_SELF_ATTN_VALUE, 0.0)

    for g in range(NB // _GRP):
        qb = sqv_ref[0, pl.ds(g * _QM, _QM), :E]
        kk = knx[pl.ds(g * _QM, _KM), :]
        logits = jax.lax.dot_general(
            qb, kk, (((1,), (1,)), ((), ())),
            preferred_element_type=jnp.float32) * temp
        lcomp = jnp.concatenate(
            [jax.lax.slice(logits, (b * BUCKET, b * BUCKET),
                           ((b + 1) * BUCKET, (b + 2) * BUCKET))
             for b in range(_GRP)], axis=0) + self_add
        mx = jnp.max(lcomp, axis=1, keepdims=True)
        p = jnp.exp(lcomp - mx)
        s = jnp.sum(p, axis=1, keepdims=True)
        inv_s = 1.0 / s
        lse = mx + jnp.log(s)
        for b in range(_GRP):
            vv = vx[pl.ds(g * _QM + b * BUCKET, 2 * BUCKET), :]
            o = jax.lax.dot(p[b * BUCKET:(b + 1) * BUCKET],
                            vv, preferred_element_type=jnp.float32)
            out_ref[0, pl.ds(g * _QM + b * BUCKET, BUCKET), :E] = (
                o * inv_s[b * BUCKET:(b + 1) * BUCKET])
        out_ref[0, pl.ds(g * _QM, _QM), E:E + 8] = jnp.broadcast_to(
            lse, (_QM, 8))


def _bucket_attention(sqv):
    return pl.pallas_call(
        _stage_c_body,
        grid=(NHB,),
        in_specs=[pl.BlockSpec((1, T, 2 * E), lambda b: (b, 0, 0))],
        out_specs=pl.BlockSpec((1, T, 2 * E), lambda b: (b, 0, 0)),
        out_shape=jax.ShapeDtypeStruct((NHB, T, 2 * E), jnp.float32),
        scratch_shapes=[
            pltpu.VMEM((T + BUCKET, E), jnp.float32),
            pltpu.VMEM((T + BUCKET, E), jnp.float32),
        ],
    )(sqv)


def _unsort_gather(ol, idx2):
    info = plsc.get_sparse_core_info()
    nw = info.num_cores * info.num_subcores
    per_w = ROWS // nw
    ch = 1024
    n_ch = per_w // ch

    @functools.partial(
        pl.kernel,
        mesh=_sc_mesh(),
        out_type=jax.ShapeDtypeStruct((ROWS, 2 * E), jnp.float32),
        scratch_types=[
            pltpu.VMEM((8, 128), jnp.int32),
            pltpu.VMEM((ch // 2, 2 * E), jnp.float32),
            pltpu.SemaphoreType.DMA,
        ],
    )
    def gather_k(ol_hbm, idx_hbm, out_hbm, idx_v, orr, sem):
        wid = lax.axis_index("s") * info.num_cores + lax.axis_index("c")
        base = wid * per_w

        def chunk(c, _):
            off = pl.multiple_of(base + c * ch, ch)
            pltpu.sync_copy(idx_hbm.at[pl.ds(pl.multiple_of(off // 128, 8), 8)],
                            idx_v)
            for half in range(2):
                hoff = pl.multiple_of(off + half * (ch // 2), ch // 2)
                cps = []
                for s in range(ch // 2 // 128):
                    row = half * (ch // 2 // 128) + s
                    cps.append(pltpu.async_copy(
                        ol_hbm.at[idx_v.at[row]],
                        orr.at[pl.ds(s * 128, 128)], sem))
                for cp in cps:
                    cp.wait()
                pltpu.sync_copy(orr, out_hbm.at[pl.ds(hoff, ch // 2)])
            return _

        lax.fori_loop(0, n_ch, chunk, None)

    return gather_k(ol, idx2)


def _stage_e_body(ol_ref, out_ref):
    l = ol_ref[:, 0, :, E:E + 1]
    m = jnp.max(l, axis=0, keepdims=True)
    w = jnp.exp(l - m)
    s = jnp.sum(w, axis=0, keepdims=True)
    probs = w / s
    out_ref[0] = jnp.sum(ol_ref[:, 0, :, :E] * probs, axis=0)


def _combine(ol_u):
    return pl.pallas_call(
        _stage_e_body,
        grid=(BH,),
        in_specs=[pl.BlockSpec((N_HASHES, 1, T, 2 * E), lambda b: (0, b, 0, 0))],
        out_specs=pl.BlockSpec((1, T, E), lambda b: (b, 0, 0)),
        out_shape=jax.ShapeDtypeStruct((BH, T, E), jnp.float32),
    )(ol_u)


def kernel(qk, k, v):
    del k
    B, t, H, e = qk.shape
    qkf = jnp.transpose(qk, (0, 2, 1, 3)).reshape(BH, T, E)
    vf = jnp.transpose(v, (0, 2, 1, 3)).reshape(BH, T, E)
    rot = jax.random.normal(jax.random.key(42), (1, E, N_HASHES, NB // 2),
                            dtype=jnp.float32)[0].reshape(E, N_HASHES * 16)

    rank = _hash_rank(qkf, rot)
    idx2 = jnp.transpose(rank, (2, 0, 1)).reshape(ROWS // 128, 128)

    qv = jnp.concatenate([qkf, vf], axis=-1).reshape(SRC_ROWS, 2 * E)
    sqv = _sort_scatter(qv, idx2)
    ol_s = _bucket_attention(sqv.reshape(NHB, T, 2 * E))
    ol_u = _unsort_gather(ol_s.reshape(ROWS, 2 * E), idx2)
    out = _combine(ol_u.reshape(N_HASHES, BH, T, 2 * E))
    return out.reshape(B, H, T, E).transpose(0, 2, 1, 3)


# speedup vs baseline: 1.3361x; 1.3361x over previous
import functools
import math

import jax
import jax.numpy as jnp
from jax import lax
from jax.experimental import pallas as pl
from jax.experimental.pallas import tpu as pltpu
from jax.experimental.pallas import tpu_sc as plsc

TOKEN_SELF_ATTN_VALUE = -5e4
BUCKET = 64
N_HASHES = 8
T = 2048
E = 64
BH = 12
NB = T // BUCKET
NHB = N_HASHES * BH
ROWS = NHB * T
SRC_ROWS = BH * T
_SPLIT = 2
_NH_S = N_HASHES // _SPLIT
ROWS_S = _NH_S * BH * T

_HIGH = jax.lax.Precision.HIGHEST


def _stage_a_body(qk_ref, rot_ref, rank_ref):
    bh = pl.program_id(0)
    x = qk_ref[0]
    r = jax.lax.dot(x, rot_ref[...],
                    preferred_element_type=jnp.float32)

    lane32 = jax.lax.broadcasted_iota(jnp.int32, (T, 32), 1)
    ohs = []
    for h in range(N_HASHES):
        rh = r[:, 16 * h:16 * h + 16]
        r2 = jnp.concatenate([rh, -rh], axis=1)
        mx = jnp.max(r2, axis=1, keepdims=True)
        cand = jnp.where(r2 >= mx, lane32, 32)
        bucket = jnp.min(cand, axis=1, keepdims=True)
        ohs.append((lane32 == bucket).astype(jnp.float32))
    oh = jnp.concatenate(ohs, axis=1)

    ri = jax.lax.broadcasted_iota(jnp.int32, (128, 128), 0)
    ci = jax.lax.broadcasted_iota(jnp.int32, (128, 128), 1)
    tri = (ri >= ci).astype(jnp.float32)
    parts = []
    offs = jnp.zeros((1, 256), jnp.float32)
    for blk in range(T // 128):
        xb = oh[128 * blk:128 * (blk + 1), :]
        inc = jax.lax.dot(tri, xb,
                          preferred_element_type=jnp.float32)
        parts.append(inc + offs)
        offs = offs + inc[127:128, :]
    csum = jnp.concatenate(parts, axis=0)
    totals = offs

    gi = jax.lax.broadcasted_iota(jnp.int32, (256, 256), 0)
    gj = jax.lax.broadcasted_iota(jnp.int32, (256, 256), 1)
    same_h = (gi // 32) == (gj // 32)
    before = jnp.logical_and(same_h, gi < gj).astype(jnp.float32)
    start = jax.lax.dot(totals, before, precision=_HIGH,
                        preferred_element_type=jnp.float32)

    sel = oh * (csum + start)
    rank = jnp.concatenate(
        [jnp.sum(sel[:, 32 * h:32 * h + 32], axis=1, keepdims=True)
         for h in range(N_HASHES)], axis=1) - 1.0

    h_iota = jax.lax.broadcasted_iota(jnp.int32, (T, N_HASHES), 1)
    base = (h_iota % _NH_S) * (BH * T) + bh * T
    rank_ref[0] = rank.astype(jnp.int32) + base


def _hash_rank(qkf, rot):
    return pl.pallas_call(
        _stage_a_body,
        grid=(BH,),
        in_specs=[
            pl.BlockSpec((1, T, E), lambda b: (b, 0, 0)),
            pl.BlockSpec((E, 16 * N_HASHES), lambda b: (0, 0)),
        ],
        out_specs=pl.BlockSpec((1, T, N_HASHES), lambda b: (b, 0, 0)),
        out_shape=jax.ShapeDtypeStruct((BH, T, N_HASHES), jnp.int32),
    )(qkf, rot)


def _sc_mesh():
    return plsc.VectorSubcoreMesh(core_axis_name="c", subcore_axis_name="s")


def _sort_scatter(qv, idx2):
    rows = idx2.shape[0] * 128
    info = plsc.get_sparse_core_info()
    nw = info.num_cores * info.num_subcores
    per_w = rows // nw
    ch = 1024
    n_ch = per_w // ch

    @functools.partial(
        pl.kernel,
        mesh=_sc_mesh(),
        out_type=jax.ShapeDtypeStruct((rows, 2 * E), jnp.float32),
        scratch_types=[
            pltpu.VMEM((8, 128), jnp.int32),
            pltpu.VMEM((ch // 2, 2 * E), jnp.float32),
            pltpu.SemaphoreType.DMA,
        ],
    )
    def scatter_k(qv_hbm, idx_hbm, out_hbm, idx_v, qr, sem):
        wid = lax.axis_index("s") * info.num_cores + lax.axis_index("c")
        base = wid * per_w
        sbase = (wid % (SRC_ROWS // per_w)) * per_w

        def chunk(c, _):
            off = pl.multiple_of(base + c * ch, ch)
            soff = pl.multiple_of(sbase + c * ch, ch)
            pltpu.sync_copy(idx_hbm.at[pl.ds(pl.multiple_of(off // 128, 8), 8)],
                            idx_v)
            for half in range(2):
                hoff = pl.multiple_of(soff + half * (ch // 2), ch // 2)
                pltpu.sync_copy(qv_hbm.at[pl.ds(hoff, ch // 2)], qr)
                cps = []
                for s in range(ch // 2 // 128):
                    row = half * (ch // 2 // 128) + s
                    cps.append(pltpu.async_copy(
                        qr.at[pl.ds(s * 128, 128)],
                        out_hbm.at[idx_v.at[row]], sem))
                for cp in cps:
                    cp.wait()
            return _

        lax.fori_loop(0, n_ch, chunk, None)

    return scatter_k(qv, idx2)


_GRP = 4
_QM = _GRP * BUCKET
_KM = (_GRP + 1) * BUCKET


def _stage_c_body(sqv_ref, out_ref, knx, vx):
    temp = 1.0 / math.sqrt(E)
    q = sqv_ref[0, :, :E]
    v = sqv_ref[0, :, E:]
    ss = jnp.sum(q * q, axis=1, keepdims=True)
    inv_norm = 1.0 / jnp.maximum(jnp.sqrt(ss), 1e-12)
    kn = q * inv_norm
    knx[pl.ds(0, BUCKET), :] = kn[T - BUCKET:, :]
    knx[pl.ds(BUCKET, T), :] = kn
    vx[pl.ds(0, BUCKET), :] = v[T - BUCKET:, :]
    vx[pl.ds(BUCKET, T), :] = v

    ii = jax.lax.broadcasted_iota(jnp.int32, (_QM, _KM), 0)
    jj = jax.lax.broadcasted_iota(jnp.int32, (_QM, _KM), 1)
    bi = ii // BUCKET
    band = jnp.logical_and(jj >= bi * BUCKET, jj < bi * BUCKET + 2 * BUCKET)
    self_mask = jj == bi * BUCKET + BUCKET + ii % BUCKET
    maskadd = jnp.where(self_mask, TOKEN_SELF_ATTN_VALUE,
                        jnp.where(band, 0.0, -1e30))

    for g in range(NB // _GRP):
        qb = sqv_ref[0, pl.ds(g * _QM, _QM), :E]
        kk = knx[pl.ds(g * _QM, _KM), :]
        vv = vx[pl.ds(g * _QM, _KM), :]
        logits = jax.lax.dot_general(
            qb, kk, (((1,), (1,)), ((), ())),
            preferred_element_type=jnp.float32) * temp + maskadd
        mx = jnp.max(logits, axis=1, keepdims=True)
        p = jnp.exp(logits - mx)
        s = jnp.sum(p, axis=1, keepdims=True)
        inv_s = 1.0 / s
        lse = mx + jnp.log(s)
        o = jax.lax.dot(p, vv,
                        preferred_element_type=jnp.float32) * inv_s
        out_ref[0, pl.ds(g * _QM, _QM), :E] = o
        out_ref[0, pl.ds(g * _QM, _QM), E:E + 8] = jnp.broadcast_to(
            lse, (_QM, 8))


def _bucket_attention(sqv):
    nhb = sqv.shape[0]
    return pl.pallas_call(
        _stage_c_body,
        grid=(nhb,),
        in_specs=[pl.BlockSpec((1, T, 2 * E), lambda b: (b, 0, 0))],
        out_specs=pl.BlockSpec((1, T, 2 * E), lambda b: (b, 0, 0)),
        out_shape=jax.ShapeDtypeStruct((nhb, T, 2 * E), jnp.float32),
        scratch_shapes=[
            pltpu.VMEM((T + BUCKET, E), jnp.float32),
            pltpu.VMEM((T + BUCKET, E), jnp.float32),
        ],
    )(sqv)


def _unsort_gather(ol, idx2):
    rows = idx2.shape[0] * 128
    info = plsc.get_sparse_core_info()
    nw = info.num_cores * info.num_subcores
    per_w = rows // nw
    ch = 1024
    n_ch = per_w // ch

    @functools.partial(
        pl.kernel,
        mesh=_sc_mesh(),
        out_type=jax.ShapeDtypeStruct((rows, 2 * E), jnp.float32),
        scratch_types=[
            pltpu.VMEM((8, 128), jnp.int32),
            pltpu.VMEM((ch // 2, 2 * E), jnp.float32),
            pltpu.SemaphoreType.DMA,
        ],
    )
    def gather_k(ol_hbm, idx_hbm, out_hbm, idx_v, orr, sem):
        wid = lax.axis_index("s") * info.num_cores + lax.axis_index("c")
        base = wid * per_w

        def chunk(c, _):
            off = pl.multiple_of(base + c * ch, ch)
            pltpu.sync_copy(idx_hbm.at[pl.ds(pl.multiple_of(off // 128, 8), 8)],
                            idx_v)
            for half in range(2):
                hoff = pl.multiple_of(off + half * (ch // 2), ch // 2)
                cps = []
                for s in range(ch // 2 // 128):
                    row = half * (ch // 2 // 128) + s
                    cps.append(pltpu.async_copy(
                        ol_hbm.at[idx_v.at[row]],
                        orr.at[pl.ds(s * 128, 128)], sem))
                for cp in cps:
                    cp.wait()
                pltpu.sync_copy(orr, out_hbm.at[pl.ds(hoff, ch // 2)])
            return _

        lax.fori_loop(0, n_ch, chunk, None)

    return gather_k(ol, idx2)


def _stage_e_body(ola_ref, olb_ref, out_ref):
    l = jnp.concatenate([ola_ref[:, 0, :, E:E + 1],
                         olb_ref[:, 0, :, E:E + 1]], axis=0)
    m = jnp.max(l, axis=0, keepdims=True)
    w = jnp.exp(l - m)
    s = jnp.sum(w, axis=0, keepdims=True)
    probs = w / s
    o = jnp.concatenate([ola_ref[:, 0, :, :E],
                         olb_ref[:, 0, :, :E]], axis=0)
    out_ref[0] = jnp.sum(o * probs, axis=0)


def _combine(ol_a, ol_b):
    spec = pl.BlockSpec((_NH_S, 1, T, 2 * E), lambda b: (0, b, 0, 0))
    return pl.pallas_call(
        _stage_e_body,
        grid=(BH,),
        in_specs=[spec, spec],
        out_specs=pl.BlockSpec((1, T, E), lambda b: (b, 0, 0)),
        out_shape=jax.ShapeDtypeStruct((BH, T, E), jnp.float32),
    )(ol_a, ol_b)


def kernel(qk, k, v):
    del k
    B, t, H, e = qk.shape
    qkf = jnp.transpose(qk, (0, 2, 1, 3)).reshape(BH, T, E)
    vf = jnp.transpose(v, (0, 2, 1, 3)).reshape(BH, T, E)
    rot = jax.random.normal(jax.random.key(42), (1, E, N_HASHES, NB // 2),
                            dtype=jnp.float32)[0].reshape(E, N_HASHES * 16)

    rank = _hash_rank(qkf, rot)
    idx2 = jnp.transpose(rank, (2, 0, 1)).reshape(ROWS // 128, 128)
    qv = jnp.concatenate([qkf, vf], axis=-1).reshape(SRC_ROWS, 2 * E)

    ol_u = []
    for s in range(_SPLIT):
        idx_s = jax.lax.slice_in_dim(idx2, s * (ROWS_S // 128),
                                     (s + 1) * (ROWS_S // 128), axis=0)
        sqv = _sort_scatter(qv, idx_s)
        ol_s = _bucket_attention(sqv.reshape(_NH_S * BH, T, 2 * E))
        ol_u.append(_unsort_gather(ol_s.reshape(ROWS_S, 2 * E), idx_s))
    out = _combine(ol_u[0].reshape(_NH_S, BH, T, 2 * E),
                   ol_u[1].reshape(_NH_S, BH, T, 2 * E))
    return out.reshape(B, H, T, E).transpose(0, 2, 1, 3)


# speedup vs baseline: 1.3746x; 1.0288x over previous
import functools
import math

import jax
import jax.numpy as jnp
from jax import lax
from jax.experimental import pallas as pl
from jax.experimental.pallas import tpu as pltpu
from jax.experimental.pallas import tpu_sc as plsc

TOKEN_SELF_ATTN_VALUE = -5e4
BUCKET = 64
N_HASHES = 8
T = 2048
E = 64
BH = 12
NB = T // BUCKET
NHB = N_HASHES * BH
ROWS = NHB * T
SRC_ROWS = BH * T
_SPLIT = 2
_NH_S = N_HASHES // _SPLIT
ROWS_S = _NH_S * BH * T

_HIGH = jax.lax.Precision.HIGHEST


_NC = _NH_S * 32


def _stage_a_body(h0, qk_ref, rot_ref, rank_ref):
    bh = pl.program_id(0)
    x = qk_ref[0]
    r = jax.lax.dot(x, rot_ref[...],
                    preferred_element_type=jnp.float32)

    lane32 = jax.lax.broadcasted_iota(jnp.int32, (T, 32), 1)
    ohs = []
    for h in range(h0, h0 + _NH_S):
        rh = r[:, 16 * h:16 * h + 16]
        r2 = jnp.concatenate([rh, -rh], axis=1)
        mx = jnp.max(r2, axis=1, keepdims=True)
        cand = jnp.where(r2 >= mx, lane32, 32)
        bucket = jnp.min(cand, axis=1, keepdims=True)
        ohs.append((lane32 == bucket).astype(jnp.float32))
    oh = jnp.concatenate(ohs, axis=1)

    ri = jax.lax.broadcasted_iota(jnp.int32, (128, 128), 0)
    ci = jax.lax.broadcasted_iota(jnp.int32, (128, 128), 1)
    tri = (ri >= ci).astype(jnp.float32)
    parts = []
    offs = jnp.zeros((1, _NC), jnp.float32)
    for blk in range(T // 128):
        xb = oh[128 * blk:128 * (blk + 1), :]
        inc = jax.lax.dot(tri, xb,
                          preferred_element_type=jnp.float32)
        parts.append(inc + offs)
        offs = offs + inc[127:128, :]
    csum = jnp.concatenate(parts, axis=0)
    totals = offs

    gi = jax.lax.broadcasted_iota(jnp.int32, (_NC, _NC), 0)
    gj = jax.lax.broadcasted_iota(jnp.int32, (_NC, _NC), 1)
    same_h = (gi // 32) == (gj // 32)
    before = jnp.logical_and(same_h, gi < gj).astype(jnp.float32)
    start = jax.lax.dot(totals, before, precision=_HIGH,
                        preferred_element_type=jnp.float32)

    sel = oh * (csum + start)
    rank = jnp.concatenate(
        [jnp.sum(sel[:, 32 * h:32 * h + 32], axis=1, keepdims=True)
         for h in range(_NH_S)], axis=1) - 1.0

    h_iota = jax.lax.broadcasted_iota(jnp.int32, (T, _NH_S), 1)
    base = h_iota * (BH * T) + bh * T
    rank_ref[0] = rank.astype(jnp.int32) + base


def _hash_rank(qkf, rot, h0):
    return pl.pallas_call(
        functools.partial(_stage_a_body, h0),
        grid=(BH,),
        in_specs=[
            pl.BlockSpec((1, T, E), lambda b: (b, 0, 0)),
            pl.BlockSpec((E, 16 * N_HASHES), lambda b: (0, 0)),
        ],
        out_specs=pl.BlockSpec((1, T, _NH_S), lambda b: (b, 0, 0)),
        out_shape=jax.ShapeDtypeStruct((BH, T, _NH_S), jnp.int32),
    )(qkf, rot)


def _sc_mesh():
    return plsc.VectorSubcoreMesh(core_axis_name="c", subcore_axis_name="s")


def _sort_scatter(qv, idx2):
    rows = idx2.shape[0] * 128
    info = plsc.get_sparse_core_info()
    nw = info.num_cores * info.num_subcores
    per_w = rows // nw
    ch = 1024
    n_ch = per_w // ch

    @functools.partial(
        pl.kernel,
        mesh=_sc_mesh(),
        out_type=jax.ShapeDtypeStruct((rows, 2 * E), jnp.float32),
        scratch_types=[
            pltpu.VMEM((8, 128), jnp.int32),
            pltpu.VMEM((ch // 2, 2 * E), jnp.float32),
            pltpu.SemaphoreType.DMA,
        ],
    )
    def scatter_k(qv_hbm, idx_hbm, out_hbm, idx_v, qr, sem):
        wid = lax.axis_index("s") * info.num_cores + lax.axis_index("c")
        base = wid * per_w
        sbase = (wid % (SRC_ROWS // per_w)) * per_w

        def chunk(c, _):
            off = pl.multiple_of(base + c * ch, ch)
            soff = pl.multiple_of(sbase + c * ch, ch)
            pltpu.sync_copy(idx_hbm.at[pl.ds(pl.multiple_of(off // 128, 8), 8)],
                            idx_v)
            for half in range(2):
                hoff = pl.multiple_of(soff + half * (ch // 2), ch // 2)
                pltpu.sync_copy(qv_hbm.at[pl.ds(hoff, ch // 2)], qr)
                cps = []
                for s in range(ch // 2 // 128):
                    row = half * (ch // 2 // 128) + s
                    cps.append(pltpu.async_copy(
                        qr.at[pl.ds(s * 128, 128)],
                        out_hbm.at[idx_v.at[row]], sem))
                for cp in cps:
                    cp.wait()
            return _

        lax.fori_loop(0, n_ch, chunk, None)

    return scatter_k(qv, idx2)


_GRP = 4
_QM = _GRP * BUCKET
_KM = (_GRP + 1) * BUCKET


def _stage_c_body(sqv_ref, out_ref, knx, vx):
    temp = 1.0 / math.sqrt(E)
    q = sqv_ref[0, :, :E]
    v = sqv_ref[0, :, E:]
    ss = jnp.sum(q * q, axis=1, keepdims=True)
    inv_norm = 1.0 / jnp.maximum(jnp.sqrt(ss), 1e-12)
    kn = q * inv_norm
    knx[pl.ds(0, BUCKET), :] = kn[T - BUCKET:, :]
    knx[pl.ds(BUCKET, T), :] = kn
    vx[pl.ds(0, BUCKET), :] = v[T - BUCKET:, :]
    vx[pl.ds(BUCKET, T), :] = v

    ii = jax.lax.broadcasted_iota(jnp.int32, (_QM, _KM), 0)
    jj = jax.lax.broadcasted_iota(jnp.int32, (_QM, _KM), 1)
    bi = ii // BUCKET
    band = jnp.logical_and(jj >= bi * BUCKET, jj < bi * BUCKET + 2 * BUCKET)
    self_mask = jj == bi * BUCKET + BUCKET + ii % BUCKET
    maskadd = jnp.where(self_mask, TOKEN_SELF_ATTN_VALUE,
                        jnp.where(band, 0.0, -1e30))

    for g in range(NB // _GRP):
        qb = sqv_ref[0, pl.ds(g * _QM, _QM), :E]
        kk = knx[pl.ds(g * _QM, _KM), :]
        vv = vx[pl.ds(g * _QM, _KM), :]
        logits = jax.lax.dot_general(
            qb, kk, (((1,), (1,)), ((), ())),
            preferred_element_type=jnp.float32) * temp + maskadd
        mx = jnp.max(logits, axis=1, keepdims=True)
        p = jnp.exp(logits - mx)
        s = jnp.sum(p, axis=1, keepdims=True)
        inv_s = 1.0 / s
        lse = mx + jnp.log(s)
        o = jax.lax.dot(p, vv,
                        preferred_element_type=jnp.float32) * inv_s
        out_ref[0, pl.ds(g * _QM, _QM), :E] = o
        out_ref[0, pl.ds(g * _QM, _QM), E:E + 8] = jnp.broadcast_to(
            lse, (_QM, 8))


def _bucket_attention(sqv):
    nhb = sqv.shape[0]
    return pl.pallas_call(
        _stage_c_body,
        grid=(nhb,),
        in_specs=[pl.BlockSpec((1, T, 2 * E), lambda b: (b, 0, 0))],
        out_specs=pl.BlockSpec((1, T, 2 * E), lambda b: (b, 0, 0)),
        out_shape=jax.ShapeDtypeStruct((nhb, T, 2 * E), jnp.float32),
        scratch_shapes=[
            pltpu.VMEM((T + BUCKET, E), jnp.float32),
            pltpu.VMEM((T + BUCKET, E), jnp.float32),
        ],
    )(sqv)


def _unsort_gather(ol, idx2):
    rows = idx2.shape[0] * 128
    info = plsc.get_sparse_core_info()
    nw = info.num_cores * info.num_subcores
    per_w = rows // nw
    ch = 1024
    n_ch = per_w // ch

    @functools.partial(
        pl.kernel,
        mesh=_sc_mesh(),
        out_type=jax.ShapeDtypeStruct((rows, 2 * E), jnp.float32),
        scratch_types=[
            pltpu.VMEM((8, 128), jnp.int32),
            pltpu.VMEM((ch // 2, 2 * E), jnp.float32),
            pltpu.SemaphoreType.DMA,
        ],
    )
    def gather_k(ol_hbm, idx_hbm, out_hbm, idx_v, orr, sem):
        wid = lax.axis_index("s") * info.num_cores + lax.axis_index("c")
        base = wid * per_w

        def chunk(c, _):
            off = pl.multiple_of(base + c * ch, ch)
            pltpu.sync_copy(idx_hbm.at[pl.ds(pl.multiple_of(off // 128, 8), 8)],
                            idx_v)
            for half in range(2):
                hoff = pl.multiple_of(off + half * (ch // 2), ch // 2)
                cps = []
                for s in range(ch // 2 // 128):
                    row = half * (ch // 2 // 128) + s
                    cps.append(pltpu.async_copy(
                        ol_hbm.at[idx_v.at[row]],
                        orr.at[pl.ds(s * 128, 128)], sem))
                for cp in cps:
                    cp.wait()
                pltpu.sync_copy(orr, out_hbm.at[pl.ds(hoff, ch // 2)])
            return _

        lax.fori_loop(0, n_ch, chunk, None)

    return gather_k(ol, idx2)


def _stage_e_body(ola_ref, olb_ref, out_ref):
    l = jnp.concatenate([ola_ref[:, 0, :, E:E + 1],
                         olb_ref[:, 0, :, E:E + 1]], axis=0)
    m = jnp.max(l, axis=0, keepdims=True)
    w = jnp.exp(l - m)
    s = jnp.sum(w, axis=0, keepdims=True)
    probs = w / s
    o = jnp.concatenate([ola_ref[:, 0, :, :E],
                         olb_ref[:, 0, :, :E]], axis=0)
    out_ref[0] = jnp.sum(o * probs, axis=0)


def _combine(ol_a, ol_b):
    spec = pl.BlockSpec((_NH_S, 1, T, 2 * E), lambda b: (0, b, 0, 0))
    return pl.pallas_call(
        _stage_e_body,
        grid=(BH,),
        in_specs=[spec, spec],
        out_specs=pl.BlockSpec((1, T, E), lambda b: (b, 0, 0)),
        out_shape=jax.ShapeDtypeStruct((BH, T, E), jnp.float32),
    )(ol_a, ol_b)


def kernel(qk, k, v):
    del k
    B, t, H, e = qk.shape
    qkf = jnp.transpose(qk, (0, 2, 1, 3)).reshape(BH, T, E)
    vf = jnp.transpose(v, (0, 2, 1, 3)).reshape(BH, T, E)
    rot = jax.random.normal(jax.random.key(42), (1, E, N_HASHES, NB // 2),
                            dtype=jnp.float32)[0].reshape(E, N_HASHES * 16)

    qv = jnp.concatenate([qkf, vf], axis=-1).reshape(SRC_ROWS, 2 * E)

    ol_u = []
    for s in range(_SPLIT):
        rank = _hash_rank(qkf, rot, s * _NH_S)
        idx_s = jnp.transpose(rank, (2, 0, 1)).reshape(ROWS_S // 128, 128)
        sqv = _sort_scatter(qv, idx_s)
        ol_s = _bucket_attention(sqv.reshape(_NH_S * BH, T, 2 * E))
        ol_u.append(_unsort_gather(ol_s.reshape(ROWS_S, 2 * E), idx_s))
    out = _combine(ol_u[0].reshape(_NH_S, BH, T, 2 * E),
                   ol_u[1].reshape(_NH_S, BH, T, 2 * E))
    return out.reshape(B, H, T, E).transpose(0, 2, 1, 3)
